# baseline (speedup 1.0000x reference)
import functools

import jax
import jax.numpy as jnp
import numpy as np
from jax import lax
from jax.experimental import pallas as pl
from jax.experimental.pallas import tpu as pltpu
from jax.experimental.pallas import tpu_sc as plsc

N = 10000
E = 320000
D = 128

NC = 2
NS = 16
DH = D // NC

CHUNK = 128
EPW = E // NS
FULL_CHUNKS = EPW // CHUNK
TAIL = EPW - FULL_CHUNKS * CHUNK
RPT = N // NS

NBUF = 4
ROUNDS = FULL_CHUNKS // NBUF


def _perm64():
    p = []
    for j in range(DH):
        g, r = divmod(j, 32)
        p.append(g * 32 + (r // 2 if r % 2 == 0 else 16 + r // 2))
    return np.array(p, dtype=np.int32)


PERM = np.concatenate([_perm64(), DH + _perm64()])


def _mm_body(x_ref, w_ref, b_ref, y_ref):
    y_ref[...] = (
        lax.dot_general(x_ref[...], w_ref[...], (((1,), (1,)), ((), ())),
                        preferred_element_type=jnp.float32)
        + b_ref[0]
    ).astype(jnp.bfloat16)


_MM_BM = 2000


def _linear(x, W, b2):
    nb = N // _MM_BM
    return pl.pallas_call(
        _mm_body,
        grid=(NC, nb),
        in_specs=[
            pl.BlockSpec((_MM_BM, D), lambda c, i: (i, 0)),
            pl.BlockSpec((DH, D), lambda c, i: (c, 0)),
            pl.BlockSpec((1, 1, DH), lambda c, i: (c, 0, 0)),
        ],
        out_specs=pl.BlockSpec((_MM_BM, DH), lambda c, i: (c * nb + i, 0)),
        out_shape=jax.ShapeDtypeStruct((NC * N, DH), jnp.bfloat16),
    )(x, W, b2)


@functools.partial(
    pl.kernel,
    mesh=plsc.VectorSubcoreMesh(core_axis_name="c", subcore_axis_name="s"),
    out_type=jax.ShapeDtypeStruct((N, D), jnp.float32),
    compiler_params=pltpu.CompilerParams(use_tc_tiling_on_sc=False,
                                         needs_layout_passes=False),
    scratch_types=[
        pltpu.VMEM((2 * NBUF, CHUNK), jnp.int32),
        pltpu.VMEM((2 * NBUF, CHUNK), jnp.int32),
        pltpu.VMEM((NBUF, CHUNK, DH), jnp.bfloat16),
        pltpu.VMEM((NBUF, CHUNK, DH), jnp.float32),
        pltpu.VMEM((TAIL,), jnp.int32),
        pltpu.VMEM((TAIL,), jnp.int32),
        pltpu.VMEM((TAIL, DH), jnp.bfloat16),
        pltpu.VMEM((TAIL, DH), jnp.float32),
        pltpu.VMEM_SHARED((N, DH), jnp.bfloat16),
        pltpu.VMEM_SHARED((N, DH), jnp.float32),
        pltpu.SemaphoreType.DMA((2 * NBUF,)),
        pltpu.SemaphoreType.DMA((NBUF,)),
        pltpu.SemaphoreType.DMA((NBUF,)),
    ],
)
def _sc_aggregate(y_hbm, ei_hbm, out_hbm,
                  sidx, didx, rows_bf, rows, tsidx, tdidx, trow_bf, trow,
                  y_sh, acc_sh, isem, gsem, ssem):
    c = lax.axis_index("c")
    s = lax.axis_index("s")
    ebase = pl.multiple_of(s * EPW, 8)

    def idx_start(i, b):
        off = pl.multiple_of(ebase + i * CHUNK, 8)
        pltpu.async_copy(ei_hbm.at[0, pl.ds(off, CHUNK)], sidx.at[b],
                         isem.at[b])
        pltpu.async_copy(ei_hbm.at[1, pl.ds(off, CHUNK)], didx.at[b],
                         isem.at[b])

    def idx_wait(i, b):
        off = pl.multiple_of(ebase + i * CHUNK, 8)
        pltpu.make_async_copy(
            ei_hbm.at[0, pl.ds(off, CHUNK)], sidx.at[b], isem.at[b]).wait()
        pltpu.make_async_copy(
            ei_hbm.at[1, pl.ds(off, CHUNK)], didx.at[b], isem.at[b]).wait()

    def convert(bf_ref, f32_ref, nrows):
        def row_body(q, _):
            for u in range(4):
                r = q * 4 + u
                for g in range(DH // 32):
                    v = bf_ref[r, pl.ds(g * 32, 32)]
                    a, bb = plsc.unpack(
                        v, format=plsc.PackFormat.INTERLEAVED,
                        preferred_element_type=jnp.float32)
                    f32_ref[r, pl.ds(g * 32, 16)] = a
                    f32_ref[r, pl.ds(g * 32 + 16, 16)] = bb
            return 0

        lax.fori_loop(0, nrows // 4, row_body, 0)

    for b in range(2 * NBUF):
        idx_start(b, b)

    def zrow(r, _):
        for k in range(DH // 16):
            rows[0, r, pl.ds(k * 16, 16)] = jnp.zeros((16,), jnp.float32)
        return 0

    lax.fori_loop(0, CHUNK, zrow, 0)
    r0 = s * RPT
    for j in range(4):
        pltpu.sync_copy(rows.at[0],
                        acc_sh.at[pl.ds(r0 + j * CHUNK, CHUNK)])
    pltpu.sync_copy(rows.at[0, pl.ds(0, RPT - 4 * CHUNK)],
                    acc_sh.at[pl.ds(r0 + 4 * CHUNK, RPT - 4 * CHUNK)])

    pltpu.sync_copy(y_hbm.at[pl.ds(c * N + r0, RPT)], y_sh.at[pl.ds(r0, RPT)])
    plsc.subcore_barrier()

    def round_body(r, _):
        outer = r * NBUF
        par = lax.rem(r, 2) * NBUF
        for b in range(NBUF):
            i = outer + b
            idx_wait(i, par + b)
            pltpu.async_copy(y_sh.at[sidx.at[par + b]], rows_bf.at[b],
                             gsem.at[b])
        for b in range(NBUF):
            pltpu.make_async_copy(
                y_sh.at[sidx.at[par + b]], rows_bf.at[b], gsem.at[b]).wait()
            convert(rows_bf.at[b], rows.at[b], CHUNK)
            pltpu.async_copy(rows.at[b], acc_sh.at[didx.at[par + b]],
                             ssem.at[b], add=True)
        for b in range(NBUF):
            i = outer + b
            pltpu.make_async_copy(
                rows.at[b], acc_sh.at[didx.at[par + b]], ssem.at[b]).wait()

            @pl.when(r < ROUNDS - 2)
            def _():
                idx_start(i + 2 * NBUF, par + b)
        return 0

    lax.fori_loop(0, ROUNDS, round_body, 0)

    toff = pl.multiple_of(ebase + FULL_CHUNKS * CHUNK, 8)
    pltpu.sync_copy(ei_hbm.at[0, pl.ds(toff, TAIL)], tsidx)
    pltpu.sync_copy(ei_hbm.at[1, pl.ds(toff, TAIL)], tdidx)
    pltpu.sync_copy(y_sh.at[tsidx], trow_bf)
    convert(trow_bf, trow, TAIL)
    pltpu.sync_copy(trow, acc_sh.at[tdidx], add=True)

    plsc.subcore_barrier()

    pltpu.sync_copy(acc_sh.at[pl.ds(r0, RPT)],
                    out_hbm.at[pl.ds(r0, RPT), pl.ds(c * DH, DH)])


def kernel(x, edge_index, W, b):
    y = _linear(x, W[PERM], b[PERM].reshape(NC, 1, DH))
    return _sc_aggregate(y, edge_index)


# speedup vs baseline: 1.6865x; 1.6865x over previous
import functools

import jax
import jax.numpy as jnp
from jax import lax
from jax.experimental import pallas as pl
from jax.experimental.pallas import tpu as pltpu
from jax.experimental.pallas import tpu_sc as plsc

N = 10000
E = 320000
D = 128

NC = 2
NS = 16
DH = D // NC

CHUNK = 128
EPW = E // NS
FULL_CHUNKS = EPW // CHUNK
TAIL = EPW - FULL_CHUNKS * CHUNK
RPT = N // NS

NBUF = 4
ROUNDS = FULL_CHUNKS // NBUF


def _mm_body(x_ref, w_ref, b_ref, y_ref):
    y_ref[...] = (
        lax.dot_general(x_ref[...], w_ref[...], (((1,), (1,)), ((), ())),
                        preferred_element_type=jnp.float32)
        + b_ref[0]
    )


_MM_BM = 1000


def _linear(x, W, b2):
    nb = N // _MM_BM
    return pl.pallas_call(
        _mm_body,
        grid=(NC, nb),
        in_specs=[
            pl.BlockSpec((_MM_BM, D), lambda c, i: (i, 0)),
            pl.BlockSpec((DH, D), lambda c, i: (c, 0)),
            pl.BlockSpec((1, 1, DH), lambda c, i: (c, 0, 0)),
        ],
        out_specs=pl.BlockSpec((_MM_BM, DH), lambda c, i: (c * nb + i, 0)),
        out_shape=jax.ShapeDtypeStruct((NC * N, DH), jnp.float32),
    )(x, W, b2)


@functools.partial(
    pl.kernel,
    mesh=plsc.VectorSubcoreMesh(core_axis_name="c", subcore_axis_name="s"),
    out_type=jax.ShapeDtypeStruct((N, D), jnp.float32),
    compiler_params=pltpu.CompilerParams(use_tc_tiling_on_sc=False),
    scratch_types=[
        pltpu.VMEM((2 * NBUF, CHUNK), jnp.int32),
        pltpu.VMEM((2 * NBUF, CHUNK), jnp.int32),
        pltpu.VMEM((NBUF, CHUNK, DH), jnp.float32),
        pltpu.VMEM((TAIL,), jnp.int32),
        pltpu.VMEM((TAIL,), jnp.int32),
        pltpu.VMEM((TAIL, DH), jnp.float32),
        pltpu.VMEM_SHARED((N, DH), jnp.float32),
        pltpu.VMEM_SHARED((N, DH), jnp.float32),
        pltpu.SemaphoreType.DMA((2 * NBUF,)),
        pltpu.SemaphoreType.DMA((NBUF,)),
        pltpu.SemaphoreType.DMA((NBUF,)),
    ],
)
def _sc_aggregate(y_hbm, ei_hbm, out_hbm,
                  sidx, didx, rows, tsidx, tdidx, trow,
                  y_sh, acc_sh, isem, gsem, ssem):
    c = lax.axis_index("c")
    s = lax.axis_index("s")
    ebase = pl.multiple_of(s * EPW, 8)

    def idx_start(i, b):
        off = pl.multiple_of(ebase + i * CHUNK, 8)
        pltpu.async_copy(ei_hbm.at[0, pl.ds(off, CHUNK)], sidx.at[b],
                         isem.at[b])
        pltpu.async_copy(ei_hbm.at[1, pl.ds(off, CHUNK)], didx.at[b],
                         isem.at[b])

    def idx_wait(i, b):
        off = pl.multiple_of(ebase + i * CHUNK, 8)
        pltpu.make_async_copy(
            ei_hbm.at[0, pl.ds(off, CHUNK)], sidx.at[b], isem.at[b]).wait()
        pltpu.make_async_copy(
            ei_hbm.at[1, pl.ds(off, CHUNK)], didx.at[b], isem.at[b]).wait()

    for b in range(2 * NBUF):
        idx_start(b, b)

    def zrow(r, _):
        for k in range(DH // 16):
            rows[0, r, pl.ds(k * 16, 16)] = jnp.zeros((16,), jnp.float32)
        return 0

    lax.fori_loop(0, CHUNK, zrow, 0)
    r0 = s * RPT
    for j in range(4):
        pltpu.sync_copy(rows.at[0],
                        acc_sh.at[pl.ds(r0 + j * CHUNK, CHUNK)])
    pltpu.sync_copy(rows.at[0, pl.ds(0, RPT - 4 * CHUNK)],
                    acc_sh.at[pl.ds(r0 + 4 * CHUNK, RPT - 4 * CHUNK)])

    pltpu.sync_copy(y_hbm.at[pl.ds(c * N + r0, RPT)], y_sh.at[pl.ds(r0, RPT)])
    plsc.subcore_barrier()

    def round_body(r, _):
        outer = r * NBUF
        par = lax.rem(r, 2) * NBUF
        for b in range(NBUF):
            i = outer + b
            idx_wait(i, par + b)
            pltpu.async_copy(y_sh.at[sidx.at[par + b]], rows.at[b],
                             gsem.at[b])
        for b in range(NBUF):
            pltpu.make_async_copy(
                y_sh.at[sidx.at[par + b]], rows.at[b], gsem.at[b]).wait()
            pltpu.async_copy(rows.at[b], acc_sh.at[didx.at[par + b]],
                             ssem.at[b], add=True)
        for b in range(NBUF):
            i = outer + b
            pltpu.make_async_copy(
                rows.at[b], acc_sh.at[didx.at[par + b]], ssem.at[b]).wait()

            @pl.when(r < ROUNDS - 2)
            def _():
                idx_start(i + 2 * NBUF, par + b)
        return 0

    lax.fori_loop(0, ROUNDS, round_body, 0)

    toff = pl.multiple_of(ebase + FULL_CHUNKS * CHUNK, 8)
    pltpu.sync_copy(ei_hbm.at[0, pl.ds(toff, TAIL)], tsidx)
    pltpu.sync_copy(ei_hbm.at[1, pl.ds(toff, TAIL)], tdidx)
    pltpu.sync_copy(y_sh.at[tsidx], trow)
    pltpu.sync_copy(trow, acc_sh.at[tdidx], add=True)

    plsc.subcore_barrier()

    pltpu.sync_copy(acc_sh.at[pl.ds(r0, RPT)],
                    out_hbm.at[pl.ds(r0, RPT), pl.ds(c * DH, DH)])


def kernel(x, edge_index, W, b):
    y = _linear(x, W, b.reshape(NC, 1, DH))
    return _sc_aggregate(y, edge_index)
